# baseline (speedup 1.0000x reference)
import functools

import jax
import jax.numpy as jnp
from jax import lax
from jax.experimental import pallas as pl
from jax.experimental.pallas import tpu as pltpu
from jax.experimental.pallas import tpu_sc as plsc

D = 32
SEQ = 50
BT = 128
NBT = 16384 // BT
B = 16384 * SEQ
TP = BT + 1
SI = 4

NC, NS = 2, 16
NW = NC * NS
NITEM = SEQ * NBT
IPW = NITEM // NW
NSUP = IPW // SI
NPAIR = NSUP // 2

_mesh = plsc.VectorSubcoreMesh(core_axis_name="c", subcore_axis_name="s")


@functools.partial(
    pl.kernel,
    out_type=jax.ShapeDtypeStruct((SEQ, 4, NBT, 8, BT), jnp.float32),
    mesh=_mesh,
    compiler_params=pltpu.CompilerParams(
        use_tc_tiling_on_sc=False, needs_layout_passes=False
    ),
    scratch_types=[
        pltpu.VMEM((IPW * BT,), jnp.int32),
        pltpu.VMEM((SI * BT, D), jnp.float32),
        pltpu.VMEM((SI * BT, D), jnp.float32),
        [pltpu.VMEM((D, TP), jnp.float32)] * SI,
        [pltpu.VMEM((D, TP), jnp.float32)] * SI,
        pltpu.SemaphoreType.DMA,
        pltpu.SemaphoreType.DMA,
        pltpu.SemaphoreType.DMA,
        pltpu.SemaphoreType.DMA,
    ],
)
def _gather_kernel(x_hbm, table_hbm, out_hbm, idx_v, rows0, rows1, tbs0,
                   tbs1, sg0, sg1, so0, so1):
    wid = lax.axis_index("s") * NC + lax.axis_index("c")
    item0 = wid * IPW
    rows = (rows0, rows1)
    tbs = (tbs0, tbs1)
    sg = (sg0, sg1)
    so = (so0, so1)

    def fire_gather(i, slot):
        for k in range(SI):
            pltpu.async_copy(
                table_hbm.at[idx_v.at[pl.ds((i * SI + k) * BT, BT)]],
                rows[slot].at[pl.ds(k * BT, BT)],
                sg[slot],
            )

    def wait_gather(slot):
        pltpu.make_async_copy(
            table_hbm.at[pl.ds(0, SI * BT)], rows[slot], sg[slot]
        ).wait()

    def fire_out(i, slot):
        for k in range(SI):
            it = item0 + i * SI + k
            s = it // NBT
            bt = it % NBT
            for dt in range(4):
                pltpu.async_copy(
                    tbs[slot][k].at[pl.ds(dt * 8, 8), pl.ds(0, BT)],
                    out_hbm.at[s, dt, bt],
                    so[slot],
                )

    def wait_out(slot):
        for _ in range(4 * SI):
            pltpu.make_async_copy(
                tbs[slot][0].at[pl.ds(0, 8), pl.ds(0, BT)],
                out_hbm.at[0, 0, 0],
                so[slot],
            ).wait()

    lanes = lax.iota(jnp.int32, 16)
    lanes16 = lanes + 16

    def transpose(slot):
        @plsc.parallel_loop(0, BT, unroll=16)
        def _(b):
            bv = jnp.full((16,), 0, jnp.int32) + b
            for k in range(SI):
                v0 = rows[slot][k * BT + b, pl.ds(0, 16)]
                v1 = rows[slot][k * BT + b, pl.ds(16, 16)]
                plsc.store_scatter(tbs[slot][k], [lanes, bv], v0)
                plsc.store_scatter(tbs[slot][k], [lanes16, bv], v1)

    pltpu.sync_copy(x_hbm.at[pl.ds(item0 * BT, IPW * BT)], idx_v)
    fire_gather(0, 0)
    fire_gather(1, 1)

    def body(p, carry):
        i = p * 2
        for sl in range(2):
            wait_gather(sl)

            @pl.when(p > 0)
            def _():
                wait_out(sl)

            transpose(sl)

            @pl.when(p < NPAIR - 1)
            def _():
                fire_gather(i + sl + 2, sl)

            fire_out(i + sl, sl)
        return carry

    lax.fori_loop(0, NPAIR, body, 0)
    wait_out(0)
    wait_out(1)


def kernel(x, table):
    xt_lin = jnp.transpose(x).reshape(B)
    out5 = _gather_kernel(xt_lin, table)
    return jnp.transpose(out5, (2, 4, 0, 1, 3)).reshape(16384, SEQ, D)


# speedup vs baseline: 1.0076x; 1.0076x over previous
import functools

import jax
import jax.numpy as jnp
from jax import lax
from jax.experimental import pallas as pl
from jax.experimental.pallas import tpu as pltpu
from jax.experimental.pallas import tpu_sc as plsc

D = 32
SEQ = 50
BT = 128
NBT = 16384 // BT
B = 16384 * SEQ
TP = BT + 1
SI = 5

NC, NS = 2, 16
NW = NC * NS
NITEM = SEQ * NBT
IPW = NITEM // NW
NSUP = IPW // SI
NPAIR = NSUP // 2

_mesh = plsc.VectorSubcoreMesh(core_axis_name="c", subcore_axis_name="s")


@functools.partial(
    pl.kernel,
    out_type=jax.ShapeDtypeStruct((SEQ, 4, NBT, 8, BT), jnp.float32),
    mesh=_mesh,
    compiler_params=pltpu.CompilerParams(
        use_tc_tiling_on_sc=False, needs_layout_passes=False
    ),
    scratch_types=[
        pltpu.VMEM((IPW * BT,), jnp.int32),
        pltpu.VMEM((SI * BT, D), jnp.float32),
        pltpu.VMEM((SI * BT, D), jnp.float32),
        [pltpu.VMEM((D, TP), jnp.float32)] * SI,
        [pltpu.VMEM((D, TP), jnp.float32)] * SI,
        pltpu.SemaphoreType.DMA,
        pltpu.SemaphoreType.DMA,
        pltpu.SemaphoreType.DMA,
        pltpu.SemaphoreType.DMA,
    ],
)
def _gather_kernel(x_hbm, table_hbm, out_hbm, idx_v, rows0, rows1, tbs0,
                   tbs1, sg0, sg1, so0, so1):
    wid = lax.axis_index("s") * NC + lax.axis_index("c")
    item0 = wid * IPW
    rows = (rows0, rows1)
    tbs = (tbs0, tbs1)
    sg = (sg0, sg1)
    so = (so0, so1)

    def fire_gather(i, slot):
        for k in range(SI):
            pltpu.async_copy(
                table_hbm.at[idx_v.at[pl.ds((i * SI + k) * BT, BT)]],
                rows[slot].at[pl.ds(k * BT, BT)],
                sg[slot],
            )

    def wait_gather(slot):
        pltpu.make_async_copy(
            table_hbm.at[pl.ds(0, SI * BT)], rows[slot], sg[slot]
        ).wait()

    def fire_out(i, slot):
        for k in range(SI):
            it = item0 + i * SI + k
            s = it // NBT
            bt = it % NBT
            for dt in range(4):
                pltpu.async_copy(
                    tbs[slot][k].at[pl.ds(dt * 8, 8), pl.ds(0, BT)],
                    out_hbm.at[s, dt, bt],
                    so[slot],
                )

    def wait_out(slot):
        for _ in range(4 * SI):
            pltpu.make_async_copy(
                tbs[slot][0].at[pl.ds(0, 8), pl.ds(0, BT)],
                out_hbm.at[0, 0, 0],
                so[slot],
            ).wait()

    lanes = lax.iota(jnp.int32, 16)
    lanes16 = lanes + 16

    def transpose(slot):
        @plsc.parallel_loop(0, BT, unroll=8)
        def _(b):
            bv = jnp.full((16,), 0, jnp.int32) + b
            for k in range(SI):
                v0 = rows[slot][k * BT + b, pl.ds(0, 16)]
                v1 = rows[slot][k * BT + b, pl.ds(16, 16)]
                plsc.store_scatter(tbs[slot][k], [lanes, bv], v0)
                plsc.store_scatter(tbs[slot][k], [lanes16, bv], v1)

    pltpu.sync_copy(x_hbm.at[pl.ds(item0 * BT, IPW * BT)], idx_v)
    fire_gather(0, 0)
    fire_gather(1, 1)

    def body(p, carry):
        i = p * 2
        for sl in range(2):
            wait_gather(sl)

            @pl.when(p > 0)
            def _():
                wait_out(sl)

            transpose(sl)

            @pl.when(p < NPAIR - 1)
            def _():
                fire_gather(i + sl + 2, sl)

            fire_out(i + sl, sl)
        return carry

    lax.fori_loop(0, NPAIR, body, 0)
    wait_out(0)
    wait_out(1)


def kernel(x, table):
    xt_lin = jnp.transpose(x).reshape(B)
    out5 = _gather_kernel(xt_lin, table)
    return jnp.transpose(out5, (2, 4, 0, 1, 3)).reshape(16384, SEQ, D)
